# baseline (speedup 1.0000x reference)
import functools

import jax
import jax.numpy as jnp
from jax import lax
from jax.experimental import pallas as pl
from jax.experimental.pallas import tpu as pltpu
from jax.experimental.pallas import tpu_sc as plsc

N = 10000
E = 320000
HP = 128
QP = 64

NC, NS = 2, 16
NW = NC * NS
EPW = E // NW
C = 80
NCHUNK = EPW // C
NP = 10240
NPT = NP // NS

BE = 3200

_F32 = jnp.float32


def _padw(w, rows, cols):
    return jnp.zeros((rows, cols), _F32).at[: w.shape[0], : w.shape[1]].set(w)


def _padv(b, n):
    return jnp.zeros((1, n), _F32).at[0, : b.shape[0]].set(b)


NBUF = 5
NBATCH = NCHUNK // NBUF
C_S = 40
NCHUNK_S = EPW // C_S
NBATCH_S = NCHUNK_S // NBUF


def _make_multigather(widths, sels, EH, CH, dtype=_F32, interpret=False):
    K = len(widths)
    need = (0 in sels, 1 in sels)
    EPWl = EH // NW
    NCHUNKl = EPWl // CH
    NBATCHl = NCHUNKl // NBUF
    assert EPWl * NW == EH and CH * NCHUNKl == EPWl
    assert NBUF * NBATCHl == NCHUNKl and CH % 8 == 0 and CH <= 128
    mesh = plsc.VectorSubcoreMesh(core_axis_name="c", subcore_axis_name="s",
                                  num_cores=NC, num_subcores=NS)
    out_type = tuple(jax.ShapeDtypeStruct((EH, w), dtype) for w in widths)
    scratch = [pltpu.VMEM((NBUF, CH), jnp.int32),
               pltpu.VMEM((NBUF, CH), jnp.int32)]
    for w in widths:
        scratch.append(pltpu.VMEM((NBUF, CH, w), dtype))
    scratch += [pltpu.SemaphoreType.DMA((2, NBUF)),
                pltpu.SemaphoreType.DMA((K, NBUF)),
                pltpu.SemaphoreType.DMA((K, NBUF))]

    @functools.partial(
        pl.kernel, out_type=out_type, mesh=mesh,
        scratch_types=tuple(scratch), interpret=interpret)
    def k(*refs):
        tables = refs[:K]
        idxs = (refs[K], refs[K + 1])
        outs = refs[K + 2: 2 * K + 2]
        ibufs = (refs[2 * K + 2], refs[2 * K + 3])
        rbufs = refs[2 * K + 4: 3 * K + 4]
        isem, gsem, wsem = refs[3 * K + 4], refs[3 * K + 5], refs[3 * K + 6]
        wid = lax.axis_index("s") * NC + lax.axis_index("c")
        base = wid * EPWl

        def outer(o, carry):
            off0 = base + o * (NBUF * CH)
            for b in range(NBUF):
                off = off0 + b * CH

                @pl.when(o > 0)
                def _(b=b, off=off):
                    for t in range(K):
                        pltpu.make_async_copy(
                            rbufs[t].at[b],
                            outs[t].at[pl.ds(off - NBUF * CH, CH)],
                            wsem.at[t, b]).wait()
                for j in range(2):
                    if need[j]:
                        pltpu.async_copy(idxs[j].at[pl.ds(off, CH)],
                                         ibufs[j].at[b], isem.at[j, b])
            gd = []
            for b in range(NBUF):
                for j in range(2):
                    if need[j]:
                        pltpu.make_async_copy(
                            idxs[j].at[pl.ds(off0 + b * CH, CH)],
                            ibufs[j].at[b], isem.at[j, b]).wait()
                gd.append([pltpu.async_copy(tables[t].at[ibufs[sels[t]].at[b]],
                                            rbufs[t].at[b], gsem.at[t, b])
                           for t in range(K)])
            for b in range(NBUF):
                off = off0 + b * CH
                for t in range(K):
                    gd[b][t].wait()
                    pltpu.async_copy(rbufs[t].at[b],
                                     outs[t].at[pl.ds(off, CH)],
                                     wsem.at[t, b])
            return carry

        lax.fori_loop(0, NBATCHl, outer, 0)
        tail = base + (NCHUNKl - NBUF) * CH
        for b in range(NBUF):
            for t in range(K):
                pltpu.make_async_copy(rbufs[t].at[b],
                                      outs[t].at[pl.ds(tail + b * CH, CH)],
                                      wsem.at[t, b]).wait()

    return k


def _make_scatter_add(EHA, EHB, CH, interpret=False):
    mesh = plsc.VectorSubcoreMesh(core_axis_name="c", subcore_axis_name="s",
                                  num_cores=NC, num_subcores=NS)

    @functools.partial(
        pl.kernel,
        out_type=jax.ShapeDtypeStruct((NC, NP, HP), _F32),
        mesh=mesh,
        scratch_types=(
            pltpu.VMEM((NBUF, CH), jnp.int32),
            pltpu.VMEM((NBUF, CH, HP), _F32),
            pltpu.VMEM_SHARED((NP, HP), _F32),
            pltpu.SemaphoreType.DMA((NBUF,)),
            pltpu.SemaphoreType.DMA((NBUF,)),
            pltpu.SemaphoreType.DMA,
        ),
        interpret=interpret)
    def k(msg_a, dst_a, msg_b, dst_b, zero_hbm, out_hbm, didx, mbufs, acc,
          isem, msem, ssem):
        cid = lax.axis_index("c")
        sid = lax.axis_index("s")
        pltpu.sync_copy(zero_hbm.at[pl.ds(sid * NPT, NPT)],
                        acc.at[pl.ds(sid * NPT, NPT)])
        plsc.subcore_barrier()

        def edge_loop(msg_hbm, dst_hbm, epw, widh):
            base = widh * epw
            nbatch = epw // CH // NBUF

            def outer(o, carry):
                off0 = base + o * (NBUF * CH)
                for b in range(NBUF):
                    off = off0 + b * CH

                    @pl.when(o > 0)
                    def _(b=b):
                        pltpu.make_async_copy(msg_hbm.at[pl.ds(0, CH)],
                                              mbufs.at[b], ssem).wait()
                    pltpu.async_copy(dst_hbm.at[pl.ds(off, CH)], didx.at[b],
                                     isem.at[b])
                    pltpu.async_copy(msg_hbm.at[pl.ds(off, CH)], mbufs.at[b],
                                     msem.at[b])
                for b in range(NBUF):
                    off = off0 + b * CH
                    pltpu.make_async_copy(dst_hbm.at[pl.ds(off, CH)],
                                          didx.at[b], isem.at[b]).wait()
                    pltpu.make_async_copy(msg_hbm.at[pl.ds(off, CH)],
                                          mbufs.at[b], msem.at[b]).wait()
                    pltpu.async_copy(mbufs.at[b], acc.at[didx.at[b]], ssem,
                                     add=True)
                return carry

            lax.fori_loop(0, nbatch, outer, 0)
            for b in range(NBUF):
                pltpu.make_async_copy(msg_hbm.at[pl.ds(0, CH)], mbufs.at[b],
                                      ssem).wait()

        @pl.when(sid < NS // 2)
        def _():
            edge_loop(msg_a, dst_a, EHA // (NS // 2) // NC,
                      sid * NC + cid)

        @pl.when(sid >= NS // 2)
        def _():
            edge_loop(msg_b, dst_b, EHB // (NS // 2) // NC,
                      (sid - NS // 2) * NC + cid)

        plsc.subcore_barrier()
        pltpu.sync_copy(acc.at[pl.ds(sid * NPT, NPT)],
                        out_hbm.at[cid, pl.ds(sid * NPT, NPT)])

    return k


def _dot(a, b):
    return jnp.dot(a, b, preferred_element_type=_F32)


def _bdot(a, b):
    return jnp.dot(a.astype(jnp.bfloat16), b.astype(jnp.bfloat16),
                   preferred_element_type=_F32)


def _node_emb_body(x_ref, w_ref, b_ref, o_ref):
    o_ref[...] = _dot(x_ref[...], w_ref[...]) + b_ref[...]


def _edge_a1_body(eattr, hs, wee, bee, wl, bl, msg_o):
    ea = _bdot(eattr[...], wee[...]) + bee[...]
    msg_o[...] = jnp.maximum(_bdot(ea, wl[...]) + bl[...] + hs[...], 0.0)


def _edge_b1a2_body(h1s, h1d, eattr, wee, bee, wc1, bc1, w1a, w1b, w2, b2,
                    wl, bl, ea_o, msg_o):
    ea0 = _bdot(eattr[...], wee[...]) + bee[...]
    t1 = _bdot(ea0, wc1[...]) + bc1[...]
    u = jnp.maximum(_bdot(h1s[...], w1a[...]) + _bdot(h1d[...], w1b[...])
                    + t1, 0.0)
    ea1 = ea0 + (_bdot(u, w2[...]) + b2[...]) * 0.5
    ea_o[...] = ea1
    msg_o[...] = jnp.maximum(_bdot(ea1, wl[...]) + bl[...] + h1s[...], 0.0)


def _edge_final_body(h2s, h2d, ea1, wc2, bc2, w1a, w1b, w2, b2, wq1, wq2,
                     wmc, bmc, wf2, bf2, wf3, bf3, o_ref):
    t2 = _bdot(ea1[...], wc2[...]) + bc2[...]
    u = jnp.maximum(_bdot(h2s[...], w1a[...]) + _bdot(h2d[...], w1b[...])
                    + t2, 0.0)
    ea2 = ea1[...] + (_bdot(u, w2[...]) + b2[...]) * 0.5
    s = _bdot(ea2, wmc[...]) + bmc[...]
    a1 = jnp.maximum(_bdot(h2s[...], wq1[...]) + _bdot(h2d[...], wq2[...])
                     + s, 0.0)
    a2 = jnp.maximum(_bdot(a1, wf2[...]) + bf2[...], 0.0)
    o_ref[...] = (jnp.sum(a2 * wf3[...], axis=1) + bf3[0, 0])[None, :]


def _node_body(agg2, h, wn1, bn1, wn2, bn2, g, b, h_o):
    out = (agg2[0] + agg2[1])[:N] + h[...]
    out = jnp.maximum(_dot(out, wn1[...]) + bn1[...], 0.0)
    out = _dot(out, wn2[...]) + bn2[...]
    mu = jnp.mean(out, axis=0, keepdims=True)
    var = jnp.mean((out - mu) ** 2, axis=0, keepdims=True)
    out = (out - mu) * lax.rsqrt(var + 1e-5) * g[...] + b[...]
    h_o[...] = (h[...] + jnp.maximum(out, 0.0)) * 0.5


def _full_spec():
    return pl.BlockSpec(memory_space=pltpu.ANY)


def _tc_single(body, n_out_shapes, interpret=False):
    return pl.pallas_call(
        body,
        out_shape=n_out_shapes,
        interpret=interpret)


def _edge_call(body, n_in_edge, widths_in, n_w, out_widths, EH, out_1d=False,
               interpret=False):
    grid = (EH // BE,)
    in_specs = []
    for w in widths_in:
        in_specs.append(pl.BlockSpec((BE, w), lambda i: (i, 0)))
    for _ in range(n_w):
        in_specs.append(pl.BlockSpec(None))
    if out_1d:
        out_specs = pl.BlockSpec((1, BE), lambda i: (0, i))
        out_shape = jax.ShapeDtypeStruct((1, EH), _F32)
    else:
        out_specs = tuple(pl.BlockSpec((BE, w), lambda i: (i, 0))
                          for (w, _) in out_widths)
        out_shape = tuple(jax.ShapeDtypeStruct((EH, w), dt)
                          for (w, dt) in out_widths)
    return pl.pallas_call(
        body, grid=grid, in_specs=in_specs, out_specs=out_specs,
        out_shape=out_shape, interpret=interpret)


def _build(interpret=False):
    EHA = 166400
    EHB = E - EHA
    CHG = 80
    CHS = 40
    g1 = {eh: _make_multigather((HP,), (0,), eh, CHG, _F32, interpret)
          for eh in (EHA, EHB)}
    g2 = {eh: _make_multigather((HP, HP), (0, 1), eh, CHG, _F32, interpret)
          for eh in (EHA, EHB)}
    scat = _make_scatter_add(EHA, EHB, CHS, interpret)
    _bf = jnp.bfloat16

    node_emb = pl.pallas_call(
        _node_emb_body, out_shape=jax.ShapeDtypeStruct((N, HP), _F32),
        interpret=interpret)

    edge_a1 = {eh: _edge_call(_edge_a1_body, 2, (16, HP), 4,
                              ((HP, _F32),), eh, interpret=interpret)
               for eh in (EHA, EHB)}
    edge_b1a2 = {eh: _edge_call(_edge_b1a2_body, 3, (HP, HP, 16), 10,
                                ((HP, _F32), (HP, _F32)), eh,
                                interpret=interpret)
                 for eh in (EHA, EHB)}
    edge_final = {eh: _edge_call(_edge_final_body, 3, (HP, HP, HP),
                                 14, None, eh, out_1d=True,
                                 interpret=interpret)
                  for eh in (EHA, EHB)}

    node_upd = pl.pallas_call(
        _node_body,
        out_shape=jax.ShapeDtypeStruct((N, HP), _F32),
        interpret=interpret)

    def run(x, edge_index, edge_attr, params):
        src = edge_index[0]
        dst = edge_index[1]
        lp = params["layers"]

        wne = _padw(params["node_emb"][0], HP, HP)
        bne = _padv(params["node_emb"][1], HP)
        wee = _padw(params["edge_emb"][0], 16, HP)
        bee = _padv(params["edge_emb"][1], HP)

        def layer_w(i):
            l = lp[i]
            wl = _padw(l["lin"][0], HP, HP)
            bl = _padv(l["lin"][1], HP)
            wn1 = _padw(l["nn1"][0], HP, HP)
            bn1 = _padv(l["nn1"][1], HP)
            wn2 = _padw(l["nn2"][0], HP, HP)
            bn2 = _padv(l["nn2"][1], HP)
            w1 = l["em1"][0]
            w1a = _padw(w1[0:100], HP, HP)
            w1b = _padw(w1[100:200], HP, HP)
            w1c = _padw(w1[200:300], HP, HP)
            b1 = _padv(l["em1"][1], HP)
            w2 = _padw(l["em2"][0], HP, HP)
            b2 = _padv(l["em2"][1], HP)
            g = _padv(l["bn_g"], HP)
            b = _padv(l["bn_b"], HP)
            return wl, bl, wn1, bn1, wn2, bn2, w1a, w1b, w1c, b1, w2, b2, g, b

        (wl1, bl1, wn1_1, bn1_1, wn2_1, bn2_1, w1a_1, w1b_1, w1c_1, b1_1,
         w2_1, b2_1, g_1, bb_1) = layer_w(0)
        (wl2, bl2, wn1_2, bn1_2, wn2_2, bn2_2, w1a_2, w1b_2, w1c_2, b1_2,
         w2_2, b2_2, g_2, bb_2) = layer_w(1)

        wm1 = params["mlp1"][0]
        wq1 = _padw(wm1[0:100], HP, QP)
        wq2 = _padw(wm1[100:200], HP, QP)
        wmc = _padw(wm1[200:300], HP, QP)
        bm1 = _padv(params["mlp1"][1], QP)
        wf2 = _padw(params["mlp2"][0], QP, QP)
        bf2 = _padv(params["mlp2"][1], QP)
        wf3 = _padv(params["mlp3"][0][:, 0], QP)
        bf3 = params["mlp3"][1].reshape(1, 1)

        zero = jnp.zeros((NP, HP), _F32)
        EHS = (EHA, EHB)
        srcs = (src[:EHA], src[EHA:])
        dsts = (dst[:EHA], dst[EHA:])
        eattrs = (edge_attr[:EHA], edge_attr[EHA:])

        h0 = node_emb(x, wne, bne)
        hs1 = [g1[EHS[i]](h0, srcs[i], dsts[i])[0] for i in range(2)]
        msg1 = [edge_a1[EHS[i]](eattrs[i], hs1[i], wee, bee, wl1, bl1)[0]
                for i in range(2)]
        agg1 = scat(msg1[0], dsts[0], msg1[1], dsts[1], zero)
        h1 = node_upd(agg1, h0, wn1_1, bn1_1, wn2_1, bn2_1, g_1, bb_1)
        hsd1 = [g2[EHS[i]](h1, h1, srcs[i], dsts[i]) for i in range(2)]
        ea1, msg2 = zip(*[
            edge_b1a2[EHS[i]](hsd1[i][0], hsd1[i][1], eattrs[i], wee, bee,
                              w1c_1, b1_1, w1a_1, w1b_1, w2_1, b2_1,
                              wl2, bl2)
            for i in range(2)])
        agg2 = scat(msg2[0], dsts[0], msg2[1], dsts[1], zero)
        h2 = node_upd(agg2, h1, wn1_2, bn1_2, wn2_2, bn2_2, g_2, bb_2)
        hsd2 = [g2[EHS[i]](h2, h2, srcs[i], dsts[i]) for i in range(2)]
        o = [edge_final[EHS[i]](hsd2[i][0], hsd2[i][1], ea1[i], w1c_2, b1_2,
                                w1a_2, w1b_2, w2_2, b2_2, wq1, wq2, wmc,
                                bm1, wf2, bf2, wf3, bf3)
             for i in range(2)]
        return jnp.concatenate([o[0].reshape(EHA), o[1].reshape(EHB)])

    return run


_run_cache = []


def kernel(x, edge_index, edge_attr, params):
    if not _run_cache:
        _run_cache.append(_build(interpret=False))
    return _run_cache[0](x, edge_index, edge_attr, params)


# speedup vs baseline: 1.0238x; 1.0238x over previous
import functools

import jax
import jax.numpy as jnp
from jax import lax
from jax.experimental import pallas as pl
from jax.experimental.pallas import tpu as pltpu
from jax.experimental.pallas import tpu_sc as plsc

N = 10000
E = 320000
HP = 128
QP = 64

NC, NS = 2, 16
NW = NC * NS
EPW = E // NW
C = 80
NCHUNK = EPW // C
NP = 10240
NPT = NP // NS

BE = 3200

_F32 = jnp.float32


def _padw(w, rows, cols):
    return jnp.zeros((rows, cols), _F32).at[: w.shape[0], : w.shape[1]].set(w)


def _padv(b, n):
    return jnp.zeros((1, n), _F32).at[0, : b.shape[0]].set(b)


NBUF = 5
NBATCH = NCHUNK // NBUF
C_S = 40
NCHUNK_S = EPW // C_S
NBATCH_S = NCHUNK_S // NBUF


def _make_multigather(widths, sels, EH, CH, dtype=_F32, interpret=False):
    K = len(widths)
    need = (0 in sels, 1 in sels)
    EPWl = EH // NW
    NCHUNKl = EPWl // CH
    NBATCHl = NCHUNKl // NBUF
    assert EPWl * NW == EH and CH * NCHUNKl == EPWl
    assert NBUF * NBATCHl == NCHUNKl and CH % 8 == 0 and CH <= 128
    mesh = plsc.VectorSubcoreMesh(core_axis_name="c", subcore_axis_name="s",
                                  num_cores=NC, num_subcores=NS)
    out_type = tuple(jax.ShapeDtypeStruct((EH, w), dtype) for w in widths)
    scratch = [pltpu.VMEM((NBUF, CH), jnp.int32),
               pltpu.VMEM((NBUF, CH), jnp.int32)]
    for w in widths:
        scratch.append(pltpu.VMEM((NBUF, CH, w), dtype))
    scratch += [pltpu.SemaphoreType.DMA((2, NBUF)),
                pltpu.SemaphoreType.DMA((K, NBUF)),
                pltpu.SemaphoreType.DMA((K, NBUF))]

    @functools.partial(
        pl.kernel, out_type=out_type, mesh=mesh,
        scratch_types=tuple(scratch), interpret=interpret)
    def k(*refs):
        tables = refs[:K]
        idxs = (refs[K], refs[K + 1])
        outs = refs[K + 2: 2 * K + 2]
        ibufs = (refs[2 * K + 2], refs[2 * K + 3])
        rbufs = refs[2 * K + 4: 3 * K + 4]
        isem, gsem, wsem = refs[3 * K + 4], refs[3 * K + 5], refs[3 * K + 6]
        wid = lax.axis_index("s") * NC + lax.axis_index("c")
        base = wid * EPWl

        def outer(o, carry):
            off0 = base + o * (NBUF * CH)
            for b in range(NBUF):
                off = off0 + b * CH

                @pl.when(o > 0)
                def _(b=b, off=off):
                    for t in range(K):
                        pltpu.make_async_copy(
                            rbufs[t].at[b],
                            outs[t].at[pl.ds(off - NBUF * CH, CH)],
                            wsem.at[t, b]).wait()
                for j in range(2):
                    if need[j]:
                        pltpu.async_copy(idxs[j].at[pl.ds(off, CH)],
                                         ibufs[j].at[b], isem.at[j, b])
            gd = []
            for b in range(NBUF):
                for j in range(2):
                    if need[j]:
                        pltpu.make_async_copy(
                            idxs[j].at[pl.ds(off0 + b * CH, CH)],
                            ibufs[j].at[b], isem.at[j, b]).wait()
                gd.append([pltpu.async_copy(tables[t].at[ibufs[sels[t]].at[b]],
                                            rbufs[t].at[b], gsem.at[t, b])
                           for t in range(K)])
            for b in range(NBUF):
                off = off0 + b * CH
                for t in range(K):
                    gd[b][t].wait()
                    pltpu.async_copy(rbufs[t].at[b],
                                     outs[t].at[pl.ds(off, CH)],
                                     wsem.at[t, b])
            return carry

        lax.fori_loop(0, NBATCHl, outer, 0)
        tail = base + (NCHUNKl - NBUF) * CH
        for b in range(NBUF):
            for t in range(K):
                pltpu.make_async_copy(rbufs[t].at[b],
                                      outs[t].at[pl.ds(tail + b * CH, CH)],
                                      wsem.at[t, b]).wait()

    return k


def _make_scatter_add(EH, CH, interpret=False):
    mesh = plsc.VectorSubcoreMesh(core_axis_name="c", subcore_axis_name="s",
                                  num_cores=NC, num_subcores=NS)

    @functools.partial(
        pl.kernel,
        out_type=jax.ShapeDtypeStruct((NC, NP, HP), _F32),
        mesh=mesh,
        scratch_types=(
            pltpu.VMEM((NBUF, CH), jnp.int32),
            pltpu.VMEM((NBUF, CH, HP), _F32),
            pltpu.VMEM_SHARED((NP, HP), _F32),
            pltpu.SemaphoreType.DMA((NBUF,)),
            pltpu.SemaphoreType.DMA((NBUF,)),
            pltpu.SemaphoreType.DMA,
        ),
        interpret=interpret)
    def k(msg_hbm, dst_hbm, zero_hbm, out_hbm, didx, mbufs, acc,
          isem, msem, ssem):
        cid = lax.axis_index("c")
        sid = lax.axis_index("s")
        wid = sid * NC + cid
        pltpu.sync_copy(zero_hbm.at[pl.ds(sid * NPT, NPT)],
                        acc.at[pl.ds(sid * NPT, NPT)])
        plsc.subcore_barrier()
        EPWl = EH // NW
        base = wid * EPWl

        def outer(o, carry):
            off0 = base + o * (NBUF * CH)
            for b in range(NBUF):
                off = off0 + b * CH

                @pl.when(o > 0)
                def _(b=b):
                    pltpu.make_async_copy(msg_hbm.at[pl.ds(0, CH)],
                                          mbufs.at[b], ssem).wait()
                pltpu.async_copy(dst_hbm.at[pl.ds(off, CH)], didx.at[b],
                                 isem.at[b])
                pltpu.async_copy(msg_hbm.at[pl.ds(off, CH)], mbufs.at[b],
                                 msem.at[b])
            for b in range(NBUF):
                off = off0 + b * CH
                pltpu.make_async_copy(dst_hbm.at[pl.ds(off, CH)],
                                      didx.at[b], isem.at[b]).wait()
                pltpu.make_async_copy(msg_hbm.at[pl.ds(off, CH)],
                                      mbufs.at[b], msem.at[b]).wait()
                pltpu.async_copy(mbufs.at[b], acc.at[didx.at[b]], ssem,
                                 add=True)
            return carry

        lax.fori_loop(0, (EH // NW) // CH // NBUF, outer, 0)
        for b in range(NBUF):
            pltpu.make_async_copy(msg_hbm.at[pl.ds(0, CH)], mbufs.at[b],
                                  ssem).wait()
        plsc.subcore_barrier()
        pltpu.sync_copy(acc.at[pl.ds(sid * NPT, NPT)],
                        out_hbm.at[cid, pl.ds(sid * NPT, NPT)])

    return k


def _dot(a, b):
    return jnp.dot(a, b, preferred_element_type=_F32)


def _bdot(a, b):
    return jnp.dot(a.astype(jnp.bfloat16), b.astype(jnp.bfloat16),
                   preferred_element_type=_F32)


def _node_emb_body(x_ref, w_ref, b_ref, o_ref):
    o_ref[...] = _dot(x_ref[...], w_ref[...]) + b_ref[...]


def _edge_a1_body(eattr, hs, wee, bee, wl, bl, msg_o):
    ea = _bdot(eattr[...], wee[...]) + bee[...]
    msg_o[...] = jnp.maximum(_bdot(ea, wl[...]) + bl[...] + hs[...], 0.0)


def _edge_b1a2_body(h1s, h1d, eattr, wee, bee, wc1, bc1, w1a, w1b, w2, b2,
                    wl, bl, ea_o, msg_o):
    ea0 = _bdot(eattr[...], wee[...]) + bee[...]
    t1 = _bdot(ea0, wc1[...]) + bc1[...]
    u = jnp.maximum(_bdot(h1s[...], w1a[...]) + _bdot(h1d[...], w1b[...])
                    + t1, 0.0)
    ea1 = ea0 + (_bdot(u, w2[...]) + b2[...]) * 0.5
    ea_o[...] = ea1
    msg_o[...] = jnp.maximum(_bdot(ea1, wl[...]) + bl[...] + h1s[...], 0.0)


def _edge_final_body(h2s, h2d, ea1, wc2, bc2, w1a, w1b, w2, b2, wq1, wq2,
                     wmc, bmc, wf2, bf2, wf3, bf3, o_ref):
    t2 = _bdot(ea1[...], wc2[...]) + bc2[...]
    u = jnp.maximum(_bdot(h2s[...], w1a[...]) + _bdot(h2d[...], w1b[...])
                    + t2, 0.0)
    ea2 = ea1[...] + (_bdot(u, w2[...]) + b2[...]) * 0.5
    s = _bdot(ea2, wmc[...]) + bmc[...]
    a1 = jnp.maximum(_bdot(h2s[...], wq1[...]) + _bdot(h2d[...], wq2[...])
                     + s, 0.0)
    a2 = jnp.maximum(_bdot(a1, wf2[...]) + bf2[...], 0.0)
    o_ref[...] = (jnp.sum(a2 * wf3[...], axis=1) + bf3[0, 0])[None, :]


def _node_body(agg2a, agg2b, h, wn1, bn1, wn2, bn2, g, b, h_o):
    out = (agg2a[0] + agg2a[1] + agg2b[0] + agg2b[1])[:N] + h[...]
    out = jnp.maximum(_dot(out, wn1[...]) + bn1[...], 0.0)
    out = _dot(out, wn2[...]) + bn2[...]
    mu = jnp.mean(out, axis=0, keepdims=True)
    var = jnp.mean((out - mu) ** 2, axis=0, keepdims=True)
    out = (out - mu) * lax.rsqrt(var + 1e-5) * g[...] + b[...]
    h_o[...] = (h[...] + jnp.maximum(out, 0.0)) * 0.5


def _full_spec():
    return pl.BlockSpec(memory_space=pltpu.ANY)


def _tc_single(body, n_out_shapes, interpret=False):
    return pl.pallas_call(
        body,
        out_shape=n_out_shapes,
        interpret=interpret)


def _edge_call(body, n_in_edge, widths_in, n_w, out_widths, EH, out_1d=False,
               interpret=False):
    grid = (EH // BE,)
    in_specs = []
    for w in widths_in:
        in_specs.append(pl.BlockSpec((BE, w), lambda i: (i, 0)))
    for _ in range(n_w):
        in_specs.append(pl.BlockSpec(None))
    if out_1d:
        out_specs = pl.BlockSpec((1, BE), lambda i: (0, i))
        out_shape = jax.ShapeDtypeStruct((1, EH), _F32)
    else:
        out_specs = tuple(pl.BlockSpec((BE, w), lambda i: (i, 0))
                          for (w, _) in out_widths)
        out_shape = tuple(jax.ShapeDtypeStruct((EH, w), dt)
                          for (w, dt) in out_widths)
    return pl.pallas_call(
        body, grid=grid, in_specs=in_specs, out_specs=out_specs,
        out_shape=out_shape, interpret=interpret)


def _build(interpret=False):
    EHA = 166400
    EHB = E - EHA
    CHG = 80
    CHS = 40
    g1 = {eh: _make_multigather((HP,), (0,), eh, CHG, _F32, interpret)
          for eh in (EHA, EHB)}
    g2 = {eh: _make_multigather((HP, HP), (0, 1), eh, CHG, _F32, interpret)
          for eh in (EHA, EHB)}
    scat = {eh: _make_scatter_add(eh, CHS, interpret) for eh in (EHA, EHB)}
    _bf = jnp.bfloat16

    node_emb = pl.pallas_call(
        _node_emb_body, out_shape=jax.ShapeDtypeStruct((N, HP), _F32),
        interpret=interpret)

    edge_a1 = {eh: _edge_call(_edge_a1_body, 2, (16, HP), 4,
                              ((HP, _F32),), eh, interpret=interpret)
               for eh in (EHA, EHB)}
    edge_b1a2 = {eh: _edge_call(_edge_b1a2_body, 3, (HP, HP, 16), 10,
                                ((HP, _F32), (HP, _F32)), eh,
                                interpret=interpret)
                 for eh in (EHA, EHB)}
    edge_final = {eh: _edge_call(_edge_final_body, 3, (HP, HP, HP),
                                 14, None, eh, out_1d=True,
                                 interpret=interpret)
                  for eh in (EHA, EHB)}

    node_upd = pl.pallas_call(
        _node_body,
        out_shape=jax.ShapeDtypeStruct((N, HP), _F32),
        interpret=interpret)

    def run(x, edge_index, edge_attr, params):
        src = edge_index[0]
        dst = edge_index[1]
        lp = params["layers"]

        wne = _padw(params["node_emb"][0], HP, HP)
        bne = _padv(params["node_emb"][1], HP)
        wee = _padw(params["edge_emb"][0], 16, HP)
        bee = _padv(params["edge_emb"][1], HP)

        def layer_w(i):
            l = lp[i]
            wl = _padw(l["lin"][0], HP, HP)
            bl = _padv(l["lin"][1], HP)
            wn1 = _padw(l["nn1"][0], HP, HP)
            bn1 = _padv(l["nn1"][1], HP)
            wn2 = _padw(l["nn2"][0], HP, HP)
            bn2 = _padv(l["nn2"][1], HP)
            w1 = l["em1"][0]
            w1a = _padw(w1[0:100], HP, HP)
            w1b = _padw(w1[100:200], HP, HP)
            w1c = _padw(w1[200:300], HP, HP)
            b1 = _padv(l["em1"][1], HP)
            w2 = _padw(l["em2"][0], HP, HP)
            b2 = _padv(l["em2"][1], HP)
            g = _padv(l["bn_g"], HP)
            b = _padv(l["bn_b"], HP)
            return wl, bl, wn1, bn1, wn2, bn2, w1a, w1b, w1c, b1, w2, b2, g, b

        (wl1, bl1, wn1_1, bn1_1, wn2_1, bn2_1, w1a_1, w1b_1, w1c_1, b1_1,
         w2_1, b2_1, g_1, bb_1) = layer_w(0)
        (wl2, bl2, wn1_2, bn1_2, wn2_2, bn2_2, w1a_2, w1b_2, w1c_2, b1_2,
         w2_2, b2_2, g_2, bb_2) = layer_w(1)

        wm1 = params["mlp1"][0]
        wq1 = _padw(wm1[0:100], HP, QP)
        wq2 = _padw(wm1[100:200], HP, QP)
        wmc = _padw(wm1[200:300], HP, QP)
        bm1 = _padv(params["mlp1"][1], QP)
        wf2 = _padw(params["mlp2"][0], QP, QP)
        bf2 = _padv(params["mlp2"][1], QP)
        wf3 = _padv(params["mlp3"][0][:, 0], QP)
        bf3 = params["mlp3"][1].reshape(1, 1)

        zero = jnp.zeros((NP, HP), _F32)
        EHS = (EHA, EHB)
        srcs = (src[:EHA], src[EHA:])
        dsts = (dst[:EHA], dst[EHA:])
        eattrs = (edge_attr[:EHA], edge_attr[EHA:])

        h0 = node_emb(x, wne, bne)
        hs1 = [g1[EHS[i]](h0, srcs[i], dsts[i])[0] for i in range(2)]
        msg1 = [edge_a1[EHS[i]](eattrs[i], hs1[i], wee, bee, wl1, bl1)[0]
                for i in range(2)]
        agg1 = [scat[EHS[i]](msg1[i], dsts[i], zero) for i in range(2)]
        h1 = node_upd(agg1[0], agg1[1], h0, wn1_1, bn1_1, wn2_1, bn2_1,
                      g_1, bb_1)
        hsd1 = [g2[EHS[i]](h1, h1, srcs[i], dsts[i]) for i in range(2)]
        ea1, msg2 = zip(*[
            edge_b1a2[EHS[i]](hsd1[i][0], hsd1[i][1], eattrs[i], wee, bee,
                              w1c_1, b1_1, w1a_1, w1b_1, w2_1, b2_1,
                              wl2, bl2)
            for i in range(2)])
        agg2 = [scat[EHS[i]](msg2[i], dsts[i], zero) for i in range(2)]
        h2 = node_upd(agg2[0], agg2[1], h1, wn1_2, bn1_2, wn2_2, bn2_2,
                      g_2, bb_2)
        hsd2 = [g2[EHS[i]](h2, h2, srcs[i], dsts[i]) for i in range(2)]
        o = [edge_final[EHS[i]](hsd2[i][0], hsd2[i][1], ea1[i], w1c_2, b1_2,
                                w1a_2, w1b_2, w2_2, b2_2, wq1, wq2, wmc,
                                bm1, wf2, bf2, wf3, bf3)
             for i in range(2)]
        return jnp.concatenate([o[0].reshape(EHA), o[1].reshape(EHB)])

    return run


_run_cache = []


def kernel(x, edge_index, edge_attr, params):
    if not _run_cache:
        _run_cache.append(_build(interpret=False))
    return _run_cache[0](x, edge_index, edge_attr, params)


# speedup vs baseline: 1.0495x; 1.0252x over previous
import functools

import jax
import jax.numpy as jnp
from jax import lax
from jax.experimental import pallas as pl
from jax.experimental.pallas import tpu as pltpu
from jax.experimental.pallas import tpu_sc as plsc

N = 10000
E = 320000
HP = 128
QP = 64

NC, NS = 2, 16
NW = NC * NS
EPW = E // NW
C = 80
NCHUNK = EPW // C
NP = 10240
NPT = NP // NS

BE = 6400

_F32 = jnp.float32


def _padw(w, rows, cols):
    return jnp.zeros((rows, cols), _F32).at[: w.shape[0], : w.shape[1]].set(w)


def _padv(b, n):
    return jnp.zeros((1, n), _F32).at[0, : b.shape[0]].set(b)


NBUF = 5
NBATCH = NCHUNK // NBUF
C_S = 40
NCHUNK_S = EPW // C_S
NBATCH_S = NCHUNK_S // NBUF


def _make_multigather(widths, sels, EH, CH, dtype=_F32, interpret=False):
    K = len(widths)
    need = (0 in sels, 1 in sels)
    EPWl = EH // NW
    NCHUNKl = EPWl // CH
    NBATCHl = NCHUNKl // NBUF
    assert EPWl * NW == EH and CH * NCHUNKl == EPWl
    assert NBUF * NBATCHl == NCHUNKl and CH % 8 == 0 and CH <= 128
    mesh = plsc.VectorSubcoreMesh(core_axis_name="c", subcore_axis_name="s",
                                  num_cores=NC, num_subcores=NS)
    out_type = tuple(jax.ShapeDtypeStruct((EH, w), dtype) for w in widths)
    scratch = [pltpu.VMEM((NBUF, CH), jnp.int32),
               pltpu.VMEM((NBUF, CH), jnp.int32)]
    for w in widths:
        scratch.append(pltpu.VMEM((NBUF, CH, w), dtype))
    scratch += [pltpu.SemaphoreType.DMA((2, NBUF)),
                pltpu.SemaphoreType.DMA((K, NBUF)),
                pltpu.SemaphoreType.DMA((K, NBUF))]

    @functools.partial(
        pl.kernel, out_type=out_type, mesh=mesh,
        scratch_types=tuple(scratch), interpret=interpret)
    def k(*refs):
        tables = refs[:K]
        idxs = (refs[K], refs[K + 1])
        outs = refs[K + 2: 2 * K + 2]
        ibufs = (refs[2 * K + 2], refs[2 * K + 3])
        rbufs = refs[2 * K + 4: 3 * K + 4]
        isem, gsem, wsem = refs[3 * K + 4], refs[3 * K + 5], refs[3 * K + 6]
        wid = lax.axis_index("s") * NC + lax.axis_index("c")
        base = wid * EPWl

        def outer(o, carry):
            off0 = base + o * (NBUF * CH)
            for b in range(NBUF):
                off = off0 + b * CH

                @pl.when(o > 0)
                def _(b=b, off=off):
                    for t in range(K):
                        pltpu.make_async_copy(
                            rbufs[t].at[b],
                            outs[t].at[pl.ds(off - NBUF * CH, CH)],
                            wsem.at[t, b]).wait()
                for j in range(2):
                    if need[j]:
                        pltpu.async_copy(idxs[j].at[pl.ds(off, CH)],
                                         ibufs[j].at[b], isem.at[j, b])
            gd = []
            for b in range(NBUF):
                for j in range(2):
                    if need[j]:
                        pltpu.make_async_copy(
                            idxs[j].at[pl.ds(off0 + b * CH, CH)],
                            ibufs[j].at[b], isem.at[j, b]).wait()
                gd.append([pltpu.async_copy(tables[t].at[ibufs[sels[t]].at[b]],
                                            rbufs[t].at[b], gsem.at[t, b])
                           for t in range(K)])
            for b in range(NBUF):
                off = off0 + b * CH
                for t in range(K):
                    gd[b][t].wait()
                    pltpu.async_copy(rbufs[t].at[b],
                                     outs[t].at[pl.ds(off, CH)],
                                     wsem.at[t, b])
            return carry

        lax.fori_loop(0, NBATCHl, outer, 0)
        tail = base + (NCHUNKl - NBUF) * CH
        for b in range(NBUF):
            for t in range(K):
                pltpu.make_async_copy(rbufs[t].at[b],
                                      outs[t].at[pl.ds(tail + b * CH, CH)],
                                      wsem.at[t, b]).wait()

    return k


def _make_scatter_add(EH, CH, interpret=False):
    mesh = plsc.VectorSubcoreMesh(core_axis_name="c", subcore_axis_name="s",
                                  num_cores=NC, num_subcores=NS)

    @functools.partial(
        pl.kernel,
        out_type=jax.ShapeDtypeStruct((NC, NP, HP), _F32),
        mesh=mesh,
        scratch_types=(
            pltpu.VMEM((NBUF, CH), jnp.int32),
            pltpu.VMEM((NBUF, CH, HP), _F32),
            pltpu.VMEM_SHARED((NP, HP), _F32),
            pltpu.SemaphoreType.DMA((NBUF,)),
            pltpu.SemaphoreType.DMA((NBUF,)),
            pltpu.SemaphoreType.DMA,
        ),
        interpret=interpret)
    def k(msg_hbm, dst_hbm, zero_hbm, out_hbm, didx, mbufs, acc,
          isem, msem, ssem):
        cid = lax.axis_index("c")
        sid = lax.axis_index("s")
        wid = sid * NC + cid
        pltpu.sync_copy(zero_hbm.at[pl.ds(sid * NPT, NPT)],
                        acc.at[pl.ds(sid * NPT, NPT)])
        plsc.subcore_barrier()
        EPWl = EH // NW
        base = wid * EPWl

        def outer(o, carry):
            off0 = base + o * (NBUF * CH)
            for b in range(NBUF):
                off = off0 + b * CH

                @pl.when(o > 0)
                def _(b=b):
                    pltpu.make_async_copy(msg_hbm.at[pl.ds(0, CH)],
                                          mbufs.at[b], ssem).wait()
                pltpu.async_copy(dst_hbm.at[pl.ds(off, CH)], didx.at[b],
                                 isem.at[b])
                pltpu.async_copy(msg_hbm.at[pl.ds(off, CH)], mbufs.at[b],
                                 msem.at[b])
            for b in range(NBUF):
                off = off0 + b * CH
                pltpu.make_async_copy(dst_hbm.at[pl.ds(off, CH)],
                                      didx.at[b], isem.at[b]).wait()
                pltpu.make_async_copy(msg_hbm.at[pl.ds(off, CH)],
                                      mbufs.at[b], msem.at[b]).wait()
                pltpu.async_copy(mbufs.at[b], acc.at[didx.at[b]], ssem,
                                 add=True)
            return carry

        lax.fori_loop(0, (EH // NW) // CH // NBUF, outer, 0)
        for b in range(NBUF):
            pltpu.make_async_copy(msg_hbm.at[pl.ds(0, CH)], mbufs.at[b],
                                  ssem).wait()
        plsc.subcore_barrier()
        pltpu.sync_copy(acc.at[pl.ds(sid * NPT, NPT)],
                        out_hbm.at[cid, pl.ds(sid * NPT, NPT)])

    return k


def _dot(a, b):
    return jnp.dot(a, b, preferred_element_type=_F32)


def _bdot(a, b):
    return jnp.dot(a.astype(jnp.bfloat16), b.astype(jnp.bfloat16),
                   preferred_element_type=_F32)


def _node_emb_body(x_ref, w_ref, b_ref, o_ref):
    o_ref[...] = _dot(x_ref[...], w_ref[...]) + b_ref[...]


def _edge_a1_body(eattr, hs, wee, bee, wl, bl, msg_o):
    ea = _bdot(eattr[...], wee[...]) + bee[...]
    msg_o[...] = jnp.maximum(_bdot(ea, wl[...]) + bl[...] + hs[...], 0.0)


def _edge_b1a2_body(h1s, h1d, eattr, wee, bee, wc1, bc1, w1a, w1b, w2, b2,
                    wl, bl, ea_o, msg_o):
    ea0 = _bdot(eattr[...], wee[...]) + bee[...]
    t1 = _bdot(ea0, wc1[...]) + bc1[...]
    u = jnp.maximum(_bdot(h1s[...], w1a[...]) + _bdot(h1d[...], w1b[...])
                    + t1, 0.0)
    ea1 = ea0 + (_bdot(u, w2[...]) + b2[...]) * 0.5
    ea_o[...] = ea1
    msg_o[...] = jnp.maximum(_bdot(ea1, wl[...]) + bl[...] + h1s[...], 0.0)


def _edge_final_body(h2s, h2d, ea1, wc2, bc2, w1a, w1b, w2, b2, wq1, wq2,
                     wmc, bmc, wf2, bf2, wf3, bf3, o_ref):
    t2 = _bdot(ea1[...], wc2[...]) + bc2[...]
    u = jnp.maximum(_bdot(h2s[...], w1a[...]) + _bdot(h2d[...], w1b[...])
                    + t2, 0.0)
    ea2 = ea1[...] + (_bdot(u, w2[...]) + b2[...]) * 0.5
    s = _bdot(ea2, wmc[...]) + bmc[...]
    a1 = jnp.maximum(_bdot(h2s[...], wq1[...]) + _bdot(h2d[...], wq2[...])
                     + s, 0.0)
    a2 = jnp.maximum(_bdot(a1, wf2[...]) + bf2[...], 0.0)
    o_ref[...] = (jnp.sum(a2 * wf3[...], axis=1) + bf3[0, 0])[None, :]


def _node_body(agg2a, agg2b, h, wn1, bn1, wn2, bn2, g, b, h_o):
    out = (agg2a[0] + agg2a[1] + agg2b[0] + agg2b[1])[:N] + h[...]
    out = jnp.maximum(_dot(out, wn1[...]) + bn1[...], 0.0)
    out = _dot(out, wn2[...]) + bn2[...]
    mu = jnp.mean(out, axis=0, keepdims=True)
    var = jnp.mean((out - mu) ** 2, axis=0, keepdims=True)
    out = (out - mu) * lax.rsqrt(var + 1e-5) * g[...] + b[...]
    h_o[...] = (h[...] + jnp.maximum(out, 0.0)) * 0.5


def _full_spec():
    return pl.BlockSpec(memory_space=pltpu.ANY)


def _tc_single(body, n_out_shapes, interpret=False):
    return pl.pallas_call(
        body,
        out_shape=n_out_shapes,
        interpret=interpret)


def _edge_call(body, n_in_edge, widths_in, n_w, out_widths, EH, out_1d=False,
               interpret=False):
    grid = (EH // BE,)
    in_specs = []
    for w in widths_in:
        in_specs.append(pl.BlockSpec((BE, w), lambda i: (i, 0)))
    for _ in range(n_w):
        in_specs.append(pl.BlockSpec(None))
    if out_1d:
        out_specs = pl.BlockSpec((1, BE), lambda i: (0, i))
        out_shape = jax.ShapeDtypeStruct((1, EH), _F32)
    else:
        out_specs = tuple(pl.BlockSpec((BE, w), lambda i: (i, 0))
                          for (w, _) in out_widths)
        out_shape = tuple(jax.ShapeDtypeStruct((EH, w), dt)
                          for (w, dt) in out_widths)
    return pl.pallas_call(
        body, grid=grid, in_specs=in_specs, out_specs=out_specs,
        out_shape=out_shape, interpret=interpret)


def _build(interpret=False):
    EHA = 166400
    EHB = E - EHA
    CHG = 80
    CHS = 40
    g1 = {eh: _make_multigather((HP,), (0,), eh, CHG, _F32, interpret)
          for eh in (EHA, EHB)}
    g2 = {eh: _make_multigather((HP, HP), (0, 1), eh, CHG, _F32, interpret)
          for eh in (EHA, EHB)}
    scat = {eh: _make_scatter_add(eh, CHS, interpret) for eh in (EHA, EHB)}
    _bf = jnp.bfloat16

    node_emb = pl.pallas_call(
        _node_emb_body, out_shape=jax.ShapeDtypeStruct((N, HP), _F32),
        interpret=interpret)

    edge_a1 = {eh: _edge_call(_edge_a1_body, 2, (16, HP), 4,
                              ((HP, _F32),), eh, interpret=interpret)
               for eh in (EHA, EHB)}
    edge_b1a2 = {eh: _edge_call(_edge_b1a2_body, 3, (HP, HP, 16), 10,
                                ((HP, _F32), (HP, _F32)), eh,
                                interpret=interpret)
                 for eh in (EHA, EHB)}
    edge_final = {eh: _edge_call(_edge_final_body, 3, (HP, HP, HP),
                                 14, None, eh, out_1d=True,
                                 interpret=interpret)
                  for eh in (EHA, EHB)}

    node_upd = pl.pallas_call(
        _node_body,
        out_shape=jax.ShapeDtypeStruct((N, HP), _F32),
        interpret=interpret)

    def run(x, edge_index, edge_attr, params):
        src = edge_index[0]
        dst = edge_index[1]
        lp = params["layers"]

        wne = _padw(params["node_emb"][0], HP, HP)
        bne = _padv(params["node_emb"][1], HP)
        wee = _padw(params["edge_emb"][0], 16, HP)
        bee = _padv(params["edge_emb"][1], HP)

        def layer_w(i):
            l = lp[i]
            wl = _padw(l["lin"][0], HP, HP)
            bl = _padv(l["lin"][1], HP)
            wn1 = _padw(l["nn1"][0], HP, HP)
            bn1 = _padv(l["nn1"][1], HP)
            wn2 = _padw(l["nn2"][0], HP, HP)
            bn2 = _padv(l["nn2"][1], HP)
            w1 = l["em1"][0]
            w1a = _padw(w1[0:100], HP, HP)
            w1b = _padw(w1[100:200], HP, HP)
            w1c = _padw(w1[200:300], HP, HP)
            b1 = _padv(l["em1"][1], HP)
            w2 = _padw(l["em2"][0], HP, HP)
            b2 = _padv(l["em2"][1], HP)
            g = _padv(l["bn_g"], HP)
            b = _padv(l["bn_b"], HP)
            return wl, bl, wn1, bn1, wn2, bn2, w1a, w1b, w1c, b1, w2, b2, g, b

        (wl1, bl1, wn1_1, bn1_1, wn2_1, bn2_1, w1a_1, w1b_1, w1c_1, b1_1,
         w2_1, b2_1, g_1, bb_1) = layer_w(0)
        (wl2, bl2, wn1_2, bn1_2, wn2_2, bn2_2, w1a_2, w1b_2, w1c_2, b1_2,
         w2_2, b2_2, g_2, bb_2) = layer_w(1)

        wm1 = params["mlp1"][0]
        wq1 = _padw(wm1[0:100], HP, QP)
        wq2 = _padw(wm1[100:200], HP, QP)
        wmc = _padw(wm1[200:300], HP, QP)
        bm1 = _padv(params["mlp1"][1], QP)
        wf2 = _padw(params["mlp2"][0], QP, QP)
        bf2 = _padv(params["mlp2"][1], QP)
        wf3 = _padv(params["mlp3"][0][:, 0], QP)
        bf3 = params["mlp3"][1].reshape(1, 1)

        zero = jnp.zeros((NP, HP), _F32)
        EHS = (EHA, EHB)
        srcs = (src[:EHA], src[EHA:])
        dsts = (dst[:EHA], dst[EHA:])
        eattrs = (edge_attr[:EHA], edge_attr[EHA:])

        h0 = node_emb(x, wne, bne)
        hs1 = [g1[EHS[i]](h0, srcs[i], dsts[i])[0] for i in range(2)]
        msg1 = [edge_a1[EHS[i]](eattrs[i], hs1[i], wee, bee, wl1, bl1)[0]
                for i in range(2)]
        agg1 = [scat[EHS[i]](msg1[i], dsts[i], zero) for i in range(2)]
        h1 = node_upd(agg1[0], agg1[1], h0, wn1_1, bn1_1, wn2_1, bn2_1,
                      g_1, bb_1)
        hsd1 = [g2[EHS[i]](h1, h1, srcs[i], dsts[i]) for i in range(2)]
        ea1, msg2 = zip(*[
            edge_b1a2[EHS[i]](hsd1[i][0], hsd1[i][1], eattrs[i], wee, bee,
                              w1c_1, b1_1, w1a_1, w1b_1, w2_1, b2_1,
                              wl2, bl2)
            for i in range(2)])
        agg2 = [scat[EHS[i]](msg2[i], dsts[i], zero) for i in range(2)]
        h2 = node_upd(agg2[0], agg2[1], h1, wn1_2, bn1_2, wn2_2, bn2_2,
                      g_2, bb_2)
        hsd2 = [g2[EHS[i]](h2, h2, srcs[i], dsts[i]) for i in range(2)]
        o = [edge_final[EHS[i]](hsd2[i][0], hsd2[i][1], ea1[i], w1c_2, b1_2,
                                w1a_2, w1b_2, w2_2, b2_2, wq1, wq2, wmc,
                                bm1, wf2, bf2, wf3, bf3)
             for i in range(2)]
        return jnp.concatenate([o[0].reshape(EHA), o[1].reshape(EHB)])

    return run


_run_cache = []


def kernel(x, edge_index, edge_attr, params):
    if not _run_cache:
        _run_cache.append(_build(interpret=False))
    return _run_cache[0](x, edge_index, edge_attr, params)


# speedup vs baseline: 1.0656x; 1.0153x over previous
import functools

import jax
import jax.numpy as jnp
from jax import lax
from jax.experimental import pallas as pl
from jax.experimental.pallas import tpu as pltpu
from jax.experimental.pallas import tpu_sc as plsc

N = 10000
E = 320000
HP = 128
QP = 64

NC, NS = 2, 16
NW = NC * NS
EPW = E // NW
C = 80
NCHUNK = EPW // C
NP = 10240
NPT = NP // NS

BE = 6400

_F32 = jnp.float32


def _padw(w, rows, cols):
    return jnp.zeros((rows, cols), _F32).at[: w.shape[0], : w.shape[1]].set(w)


def _padv(b, n):
    return jnp.zeros((1, n), _F32).at[0, : b.shape[0]].set(b)


NBUF = 5
NBATCH = NCHUNK // NBUF
C_S = 40
NCHUNK_S = EPW // C_S
NBATCH_S = NCHUNK_S // NBUF


def _make_multigather(widths, sels, EH, CH, dtype=_F32, interpret=False):
    K = len(widths)
    need = (0 in sels, 1 in sels)
    EPWl = EH // NW
    NCHUNKl = EPWl // CH
    NBATCHl = NCHUNKl // NBUF
    assert EPWl * NW == EH and CH * NCHUNKl == EPWl
    assert NBUF * NBATCHl == NCHUNKl and CH % 8 == 0 and CH <= 128
    mesh = plsc.VectorSubcoreMesh(core_axis_name="c", subcore_axis_name="s",
                                  num_cores=NC, num_subcores=NS)
    out_type = tuple(jax.ShapeDtypeStruct((EH, w), dtype) for w in widths)
    scratch = [pltpu.VMEM((NBUF, CH), jnp.int32),
               pltpu.VMEM((NBUF, CH), jnp.int32)]
    for w in widths:
        scratch.append(pltpu.VMEM((NBUF, CH, w), dtype))
    scratch += [pltpu.SemaphoreType.DMA((2, NBUF)),
                pltpu.SemaphoreType.DMA((K, NBUF)),
                pltpu.SemaphoreType.DMA((K, NBUF))]

    @functools.partial(
        pl.kernel, out_type=out_type, mesh=mesh,
        scratch_types=tuple(scratch), interpret=interpret)
    def k(*refs):
        tables = refs[:K]
        idxs = (refs[K], refs[K + 1])
        outs = refs[K + 2: 2 * K + 2]
        ibufs = (refs[2 * K + 2], refs[2 * K + 3])
        rbufs = refs[2 * K + 4: 3 * K + 4]
        isem, gsem, wsem = refs[3 * K + 4], refs[3 * K + 5], refs[3 * K + 6]
        wid = lax.axis_index("s") * NC + lax.axis_index("c")
        base = wid * EPWl

        def outer(o, carry):
            off0 = base + o * (NBUF * CH)
            for b in range(NBUF):
                off = off0 + b * CH

                @pl.when(o > 0)
                def _(b=b, off=off):
                    for t in range(K):
                        pltpu.make_async_copy(
                            rbufs[t].at[b],
                            outs[t].at[pl.ds(off - NBUF * CH, CH)],
                            wsem.at[t, b]).wait()
                for j in range(2):
                    if need[j]:
                        pltpu.async_copy(idxs[j].at[pl.ds(off, CH)],
                                         ibufs[j].at[b], isem.at[j, b])
            gd = []
            for b in range(NBUF):
                for j in range(2):
                    if need[j]:
                        pltpu.make_async_copy(
                            idxs[j].at[pl.ds(off0 + b * CH, CH)],
                            ibufs[j].at[b], isem.at[j, b]).wait()
                gd.append([pltpu.async_copy(tables[t].at[ibufs[sels[t]].at[b]],
                                            rbufs[t].at[b], gsem.at[t, b])
                           for t in range(K)])
            for b in range(NBUF):
                off = off0 + b * CH
                for t in range(K):
                    gd[b][t].wait()
                    pltpu.async_copy(rbufs[t].at[b],
                                     outs[t].at[pl.ds(off, CH)],
                                     wsem.at[t, b])
            return carry

        lax.fori_loop(0, NBATCHl, outer, 0)
        tail = base + (NCHUNKl - NBUF) * CH
        for b in range(NBUF):
            for t in range(K):
                pltpu.make_async_copy(rbufs[t].at[b],
                                      outs[t].at[pl.ds(tail + b * CH, CH)],
                                      wsem.at[t, b]).wait()

    return k


def _make_scatter_add(EH, CH, interpret=False):
    mesh = plsc.VectorSubcoreMesh(core_axis_name="c", subcore_axis_name="s",
                                  num_cores=NC, num_subcores=NS)

    @functools.partial(
        pl.kernel,
        out_type=jax.ShapeDtypeStruct((NC, NP, HP), _F32),
        mesh=mesh,
        scratch_types=(
            pltpu.VMEM((NBUF, CH), jnp.int32),
            pltpu.VMEM((NBUF, CH, HP), _F32),
            pltpu.VMEM_SHARED((NP, HP), _F32),
            pltpu.SemaphoreType.DMA((NBUF,)),
            pltpu.SemaphoreType.DMA((NBUF,)),
            pltpu.SemaphoreType.DMA,
        ),
        interpret=interpret)
    def k(msg_hbm, dst_hbm, zero_hbm, out_hbm, didx, mbufs, acc,
          isem, msem, ssem):
        cid = lax.axis_index("c")
        sid = lax.axis_index("s")
        wid = sid * NC + cid
        pltpu.sync_copy(zero_hbm.at[pl.ds(sid * NPT, NPT)],
                        acc.at[pl.ds(sid * NPT, NPT)])
        plsc.subcore_barrier()
        EPWl = EH // NW
        base = wid * EPWl

        def outer(o, carry):
            off0 = base + o * (NBUF * CH)
            for b in range(NBUF):
                off = off0 + b * CH

                @pl.when(o > 0)
                def _(b=b):
                    pltpu.make_async_copy(msg_hbm.at[pl.ds(0, CH)],
                                          mbufs.at[b], ssem).wait()
                pltpu.async_copy(dst_hbm.at[pl.ds(off, CH)], didx.at[b],
                                 isem.at[b])
                pltpu.async_copy(msg_hbm.at[pl.ds(off, CH)], mbufs.at[b],
                                 msem.at[b])
            for b in range(NBUF):
                off = off0 + b * CH
                pltpu.make_async_copy(dst_hbm.at[pl.ds(off, CH)],
                                      didx.at[b], isem.at[b]).wait()
                pltpu.make_async_copy(msg_hbm.at[pl.ds(off, CH)],
                                      mbufs.at[b], msem.at[b]).wait()
                pltpu.async_copy(mbufs.at[b], acc.at[didx.at[b]], ssem,
                                 add=True)
            return carry

        lax.fori_loop(0, (EH // NW) // CH // NBUF, outer, 0)
        for b in range(NBUF):
            pltpu.make_async_copy(msg_hbm.at[pl.ds(0, CH)], mbufs.at[b],
                                  ssem).wait()
        plsc.subcore_barrier()
        pltpu.sync_copy(acc.at[pl.ds(sid * NPT, NPT)],
                        out_hbm.at[cid, pl.ds(sid * NPT, NPT)])

    return k


def _dot(a, b):
    return jnp.dot(a, b, preferred_element_type=_F32)


def _bdot(a, b):
    return jnp.dot(a.astype(jnp.bfloat16), b.astype(jnp.bfloat16),
                   preferred_element_type=_F32)


def _node_emb_body(x_ref, w_ref, b_ref, o_ref):
    o_ref[...] = _dot(x_ref[...], w_ref[...]) + b_ref[...]


def _edge_a1_body(eattr, hs, wee, bee, wl, bl, msg_o):
    ea = _bdot(eattr[...], wee[...]) + bee[...]
    msg_o[...] = jnp.maximum(_bdot(ea, wl[...]) + bl[...] + hs[...], 0.0)


def _edge_b1a2_body(h1s, h1d, eattr, wee, bee, wc1, bc1, w1a, w1b, w2, b2,
                    wl, bl, ea_o, msg_o):
    ea0 = _bdot(eattr[...], wee[...]) + bee[...]
    t1 = _bdot(ea0, wc1[...]) + bc1[...]
    u = jnp.maximum(_bdot(h1s[...], w1a[...]) + _bdot(h1d[...], w1b[...])
                    + t1, 0.0)
    ea1 = ea0 + (_bdot(u, w2[...]) + b2[...]) * 0.5
    ea_o[...] = ea1.astype(jnp.bfloat16)
    msg_o[...] = jnp.maximum(_bdot(ea1, wl[...]) + bl[...] + h1s[...], 0.0)


def _edge_final_body(h2s, h2d, ea1, wc2, bc2, w1a, w1b, w2, b2, wq1, wq2,
                     wmc, bmc, wf2, bf2, wf3, bf3, o_ref):
    t2 = _bdot(ea1[...], wc2[...]) + bc2[...]
    u = jnp.maximum(_bdot(h2s[...], w1a[...]) + _bdot(h2d[...], w1b[...])
                    + t2, 0.0)
    ea2 = ea1[...] + (_bdot(u, w2[...]) + b2[...]) * 0.5
    s = _bdot(ea2, wmc[...]) + bmc[...]
    a1 = jnp.maximum(_bdot(h2s[...], wq1[...]) + _bdot(h2d[...], wq2[...])
                     + s, 0.0)
    a2 = jnp.maximum(_bdot(a1, wf2[...]) + bf2[...], 0.0)
    o_ref[...] = (jnp.sum(a2 * wf3[...], axis=1) + bf3[0, 0])[None, :]


def _node_body(agg2a, agg2b, h, wn1, bn1, wn2, bn2, g, b, h_o):
    out = (agg2a[0] + agg2a[1] + agg2b[0] + agg2b[1])[:N] + h[...]
    out = jnp.maximum(_dot(out, wn1[...]) + bn1[...], 0.0)
    out = _dot(out, wn2[...]) + bn2[...]
    mu = jnp.mean(out, axis=0, keepdims=True)
    var = jnp.mean((out - mu) ** 2, axis=0, keepdims=True)
    out = (out - mu) * lax.rsqrt(var + 1e-5) * g[...] + b[...]
    h_o[...] = (h[...] + jnp.maximum(out, 0.0)) * 0.5


def _full_spec():
    return pl.BlockSpec(memory_space=pltpu.ANY)


def _tc_single(body, n_out_shapes, interpret=False):
    return pl.pallas_call(
        body,
        out_shape=n_out_shapes,
        interpret=interpret)


def _edge_call(body, n_in_edge, widths_in, n_w, out_widths, EH, out_1d=False,
               interpret=False):
    grid = (EH // BE,)
    in_specs = []
    for w in widths_in:
        in_specs.append(pl.BlockSpec((BE, w), lambda i: (i, 0)))
    for _ in range(n_w):
        in_specs.append(pl.BlockSpec(None))
    if out_1d:
        out_specs = pl.BlockSpec((1, BE), lambda i: (0, i))
        out_shape = jax.ShapeDtypeStruct((1, EH), _F32)
    else:
        out_specs = tuple(pl.BlockSpec((BE, w), lambda i: (i, 0))
                          for (w, _) in out_widths)
        out_shape = tuple(jax.ShapeDtypeStruct((EH, w), dt)
                          for (w, dt) in out_widths)
    return pl.pallas_call(
        body, grid=grid, in_specs=in_specs, out_specs=out_specs,
        out_shape=out_shape, interpret=interpret)


def _build(interpret=False):
    EHA = 166400
    EHB = E - EHA
    CHG = 80
    CHS = 40
    g1 = {eh: _make_multigather((HP,), (0,), eh, CHG, _F32, interpret)
          for eh in (EHA, EHB)}
    g2 = {eh: _make_multigather((HP, HP), (0, 1), eh, CHG, _F32, interpret)
          for eh in (EHA, EHB)}
    scat = {eh: _make_scatter_add(eh, CHS, interpret) for eh in (EHA, EHB)}
    _bf = jnp.bfloat16

    node_emb = pl.pallas_call(
        _node_emb_body, out_shape=jax.ShapeDtypeStruct((N, HP), _F32),
        interpret=interpret)

    edge_a1 = {eh: _edge_call(_edge_a1_body, 2, (16, HP), 4,
                              ((HP, _F32),), eh, interpret=interpret)
               for eh in (EHA, EHB)}
    edge_b1a2 = {eh: _edge_call(_edge_b1a2_body, 3, (HP, HP, 16), 10,
                                ((HP, _bf), (HP, _F32)), eh,
                                interpret=interpret)
                 for eh in (EHA, EHB)}
    edge_final = {eh: _edge_call(_edge_final_body, 3, (HP, HP, HP),
                                 14, None, eh, out_1d=True,
                                 interpret=interpret)
                  for eh in (EHA, EHB)}

    node_upd = pl.pallas_call(
        _node_body,
        out_shape=jax.ShapeDtypeStruct((N, HP), _F32),
        interpret=interpret)

    def run(x, edge_index, edge_attr, params):
        src = edge_index[0]
        dst = edge_index[1]
        lp = params["layers"]

        wne = _padw(params["node_emb"][0], HP, HP)
        bne = _padv(params["node_emb"][1], HP)
        wee = _padw(params["edge_emb"][0], 16, HP)
        bee = _padv(params["edge_emb"][1], HP)

        def layer_w(i):
            l = lp[i]
            wl = _padw(l["lin"][0], HP, HP)
            bl = _padv(l["lin"][1], HP)
            wn1 = _padw(l["nn1"][0], HP, HP)
            bn1 = _padv(l["nn1"][1], HP)
            wn2 = _padw(l["nn2"][0], HP, HP)
            bn2 = _padv(l["nn2"][1], HP)
            w1 = l["em1"][0]
            w1a = _padw(w1[0:100], HP, HP)
            w1b = _padw(w1[100:200], HP, HP)
            w1c = _padw(w1[200:300], HP, HP)
            b1 = _padv(l["em1"][1], HP)
            w2 = _padw(l["em2"][0], HP, HP)
            b2 = _padv(l["em2"][1], HP)
            g = _padv(l["bn_g"], HP)
            b = _padv(l["bn_b"], HP)
            return wl, bl, wn1, bn1, wn2, bn2, w1a, w1b, w1c, b1, w2, b2, g, b

        (wl1, bl1, wn1_1, bn1_1, wn2_1, bn2_1, w1a_1, w1b_1, w1c_1, b1_1,
         w2_1, b2_1, g_1, bb_1) = layer_w(0)
        (wl2, bl2, wn1_2, bn1_2, wn2_2, bn2_2, w1a_2, w1b_2, w1c_2, b1_2,
         w2_2, b2_2, g_2, bb_2) = layer_w(1)

        wm1 = params["mlp1"][0]
        wq1 = _padw(wm1[0:100], HP, QP)
        wq2 = _padw(wm1[100:200], HP, QP)
        wmc = _padw(wm1[200:300], HP, QP)
        bm1 = _padv(params["mlp1"][1], QP)
        wf2 = _padw(params["mlp2"][0], QP, QP)
        bf2 = _padv(params["mlp2"][1], QP)
        wf3 = _padv(params["mlp3"][0][:, 0], QP)
        bf3 = params["mlp3"][1].reshape(1, 1)

        zero = jnp.zeros((NP, HP), _F32)
        EHS = (EHA, EHB)
        srcs = (src[:EHA], src[EHA:])
        dsts = (dst[:EHA], dst[EHA:])
        eattrs = (edge_attr[:EHA], edge_attr[EHA:])

        h0 = node_emb(x, wne, bne)
        hs1 = [g1[EHS[i]](h0, srcs[i], dsts[i])[0] for i in range(2)]
        msg1 = [edge_a1[EHS[i]](eattrs[i], hs1[i], wee, bee, wl1, bl1)[0]
                for i in range(2)]
        agg1 = [scat[EHS[i]](msg1[i], dsts[i], zero) for i in range(2)]
        h1 = node_upd(agg1[0], agg1[1], h0, wn1_1, bn1_1, wn2_1, bn2_1,
                      g_1, bb_1)
        hsd1 = [g2[EHS[i]](h1, h1, srcs[i], dsts[i]) for i in range(2)]
        ea1, msg2 = zip(*[
            edge_b1a2[EHS[i]](hsd1[i][0], hsd1[i][1], eattrs[i], wee, bee,
                              w1c_1, b1_1, w1a_1, w1b_1, w2_1, b2_1,
                              wl2, bl2)
            for i in range(2)])
        agg2 = [scat[EHS[i]](msg2[i], dsts[i], zero) for i in range(2)]
        h2 = node_upd(agg2[0], agg2[1], h1, wn1_2, bn1_2, wn2_2, bn2_2,
                      g_2, bb_2)
        hsd2 = [g2[EHS[i]](h2, h2, srcs[i], dsts[i]) for i in range(2)]
        o = [edge_final[EHS[i]](hsd2[i][0], hsd2[i][1], ea1[i], w1c_2, b1_2,
                                w1a_2, w1b_2, w2_2, b2_2, wq1, wq2, wmc,
                                bm1, wf2, bf2, wf3, bf3)
             for i in range(2)]
        return jnp.concatenate([o[0].reshape(EHA), o[1].reshape(EHB)])

    return run


_run_cache = []


def kernel(x, edge_index, edge_attr, params):
    if not _run_cache:
        _run_cache.append(_build(interpret=False))
    return _run_cache[0](x, edge_index, edge_attr, params)
